# baseline (speedup 1.0000x reference)
import functools

import jax
import jax.numpy as jnp
from jax import lax
from jax.experimental import pallas as pl
from jax.experimental.pallas import tpu as pltpu
from jax.experimental.pallas import tpu_sc as plsc

LANES = 16
TGROUPS = 4
SGROUPS = 8
SUBS = 9


def _make_sc_kernel(T, S, D):
    info = plsc.get_sparse_core_info()
    NC, NS = info.num_cores, info.num_subcores
    assert NC * NS == TGROUPS * SGROUPS
    assert T % TGROUPS == 0 and S % SGROUPS == 0 and D % LANES == 0
    tpg = T // TGROUPS
    rows = S // SGROUPS
    assert rows % SUBS == 0
    srows = rows // SUBS
    assert srows % 8 == 0 and rows % 8 == 0 and tpg % 8 == 0
    dv = D // LANES

    mesh = plsc.VectorSubcoreMesh(core_axis_name="c", subcore_axis_name="s")

    @functools.partial(
        pl.kernel,
        mesh=mesh,
        out_type=jax.ShapeDtypeStruct((T, S, D), jnp.float32),
        scratch_types=(
            [pltpu.VMEM((rows, D), jnp.float32),
             pltpu.VMEM((tpg, D), jnp.float32)]
            + [pltpu.VMEM((srows, D), jnp.float32)] * SUBS
            + [pltpu.SemaphoreType.DMA] * SUBS
        ),
    )
    def k(spatial_hbm, temporal_hbm, out_hbm, sp_v, tq_v, *ring):
        wid = lax.axis_index("s") * NC + lax.axis_index("c")
        ti = wid // SGROUPS
        si = wid - ti * SGROUPS
        s_base = pl.multiple_of(si * rows, 8)
        t_base = pl.multiple_of(ti * tpg, 8)
        bufs = ring[:SUBS]
        sems = ring[SUBS:]

        def sp_in(sub):
            s_off = pl.multiple_of(s_base + sub * srows, 8)
            return pltpu.make_async_copy(
                spatial_hbm.at[pl.ds(s_off, srows), :],
                sp_v.at[pl.ds(sub * srows, srows), :],
                sems[sub],
            )

        for sub in range(SUBS):
            sp_in(sub).start()
        pltpu.sync_copy(temporal_hbm.at[pl.ds(t_base, tpg), :], tq_v)

        def compute(tl, sub, buf):
            @plsc.parallel_loop(0, dv, unroll=2)
            def body(j):
                col = pl.multiple_of(j * LANES, LANES)
                tvec = tq_v[tl, pl.ds(col, LANES)]
                for r in range(srows):
                    buf[r, pl.ds(col, LANES)] = (
                        sp_v[sub * srows + r, pl.ds(col, LANES)] + tvec
                    )

        def out_dma(tl, sub, buf, sem):
            s_off = pl.multiple_of(s_base + sub * srows, 8)
            return pltpu.make_async_copy(
                buf, out_hbm.at[t_base + tl, pl.ds(s_off, srows), :], sem
            )

        for sub in range(SUBS):
            sp_in(sub).wait()
            compute(0, sub, bufs[sub])
            out_dma(0, sub, bufs[sub], sems[sub]).start()

        @pl.loop(1, tpg)
        def t_loop(tl):
            for sub in range(SUBS):
                out_dma(tl - 1, sub, bufs[sub], sems[sub]).wait()
                compute(tl, sub, bufs[sub])
                out_dma(tl, sub, bufs[sub], sems[sub]).start()

        for sub in range(SUBS):
            out_dma(tpg - 1, sub, bufs[sub], sems[sub]).wait()

    return k


@jax.jit
def kernel(x, spatial_table, temporal_table):
    T = x.shape[1]
    S = x.shape[2]
    D = spatial_table.shape[1]
    k = _make_sc_kernel(T, S, D)
    return k(spatial_table[:S].astype(jnp.float32),
             temporal_table.astype(jnp.float32))


# speedup vs baseline: 1.0089x; 1.0089x over previous
import functools

import jax
import jax.numpy as jnp
from jax import lax
from jax.experimental import pallas as pl
from jax.experimental.pallas import tpu as pltpu
from jax.experimental.pallas import tpu_sc as plsc

LANES = 16
TGROUPS = 4
SGROUPS = 8
SUBS = 9


def _make_sc_kernel(T, S, D):
    info = plsc.get_sparse_core_info()
    NC, NS = info.num_cores, info.num_subcores
    assert NC * NS == TGROUPS * SGROUPS
    assert T % TGROUPS == 0 and S % SGROUPS == 0 and D % LANES == 0
    tpg = T // TGROUPS
    rows = S // SGROUPS
    assert rows % SUBS == 0
    srows = rows // SUBS
    assert srows % 8 == 0 and rows % 8 == 0 and tpg % 8 == 0
    dv = D // LANES

    mesh = plsc.VectorSubcoreMesh(core_axis_name="c", subcore_axis_name="s")

    @functools.partial(
        pl.kernel,
        mesh=mesh,
        out_type=jax.ShapeDtypeStruct((T, S, D), jnp.float32),
        scratch_types=(
            [pltpu.VMEM((rows, D), jnp.float32),
             pltpu.VMEM((tpg, D), jnp.float32)]
            + [pltpu.VMEM((srows, D), jnp.float32)] * SUBS
            + [pltpu.SemaphoreType.DMA] * SUBS
        ),
    )
    def k(spatial_hbm, temporal_hbm, out_hbm, sp_v, tq_v, *ring):
        wid = lax.axis_index("s") * NC + lax.axis_index("c")
        ti = wid // SGROUPS
        si = wid - ti * SGROUPS
        s_base = pl.multiple_of(si * rows, 8)
        t_base = pl.multiple_of(ti * tpg, 8)
        bufs = ring[:SUBS]
        sems = ring[SUBS:]

        sp_in = pltpu.make_async_copy(
            spatial_hbm.at[pl.ds(s_base, rows), :], sp_v, sems[0])
        tq_in = pltpu.make_async_copy(
            temporal_hbm.at[pl.ds(t_base, tpg), :], tq_v, sems[1])
        sp_in.start()
        tq_in.start()
        tq_in.wait()
        sp_in.wait()

        def compute(tl, sub, buf):
            @plsc.parallel_loop(0, dv, unroll=2)
            def body(j):
                col = pl.multiple_of(j * LANES, LANES)
                tvec = tq_v[tl, pl.ds(col, LANES)]
                for r in range(srows):
                    buf[r, pl.ds(col, LANES)] = (
                        sp_v[sub * srows + r, pl.ds(col, LANES)] + tvec
                    )

        def out_dma(tl, sub, buf, sem):
            s_off = pl.multiple_of(s_base + sub * srows, 8)
            return pltpu.make_async_copy(
                buf, out_hbm.at[t_base + tl, pl.ds(s_off, srows), :], sem
            )

        for sub in range(SUBS):
            compute(0, sub, bufs[sub])
            out_dma(0, sub, bufs[sub], sems[sub]).start()

        @pl.loop(1, tpg)
        def t_loop(tl):
            for sub in range(SUBS):
                out_dma(tl - 1, sub, bufs[sub], sems[sub]).wait()
                compute(tl, sub, bufs[sub])
                out_dma(tl, sub, bufs[sub], sems[sub]).start()

        for sub in range(SUBS):
            out_dma(tpg - 1, sub, bufs[sub], sems[sub]).wait()

    return k


@jax.jit
def kernel(x, spatial_table, temporal_table):
    T = x.shape[1]
    S = x.shape[2]
    D = spatial_table.shape[1]
    k = _make_sc_kernel(T, S, D)
    return k(spatial_table[:S].astype(jnp.float32),
             temporal_table.astype(jnp.float32))


# speedup vs baseline: 1.0496x; 1.0404x over previous
import functools

import jax
import jax.numpy as jnp
from jax import lax
from jax.experimental import pallas as pl
from jax.experimental.pallas import tpu as pltpu
from jax.experimental.pallas import tpu_sc as plsc

LANES = 16
TGROUPS = 4
SGROUPS = 8
SUBS = 9


def _make_sc_kernel(T, S, D):
    info = plsc.get_sparse_core_info()
    NC, NS = info.num_cores, info.num_subcores
    assert NC * NS == TGROUPS * SGROUPS
    assert T % TGROUPS == 0 and S % SGROUPS == 0 and D % LANES == 0
    tpg = T // TGROUPS
    rows = S // SGROUPS
    assert rows % SUBS == 0
    srows = rows // SUBS
    assert srows % 8 == 0 and rows % 8 == 0 and tpg % 8 == 0
    dv = D // LANES

    mesh = plsc.VectorSubcoreMesh(core_axis_name="c", subcore_axis_name="s")

    @functools.partial(
        pl.kernel,
        mesh=mesh,
        out_type=jax.ShapeDtypeStruct((T, S, D), jnp.float32),
        scratch_types=(
            [pltpu.VMEM((rows, D), jnp.float32),
             pltpu.VMEM((tpg, D), jnp.float32)]
            + [pltpu.VMEM((srows, D), jnp.float32)] * SUBS
            + [pltpu.SemaphoreType.DMA] * SUBS
        ),
    )
    def k(spatial_hbm, temporal_hbm, out_hbm, sp_v, tq_v, *ring):
        wid = lax.axis_index("s") * NC + lax.axis_index("c")
        ti = wid // SGROUPS
        si = wid - ti * SGROUPS
        s_base = pl.multiple_of(si * rows, 8)
        t_base = pl.multiple_of(ti * tpg, 8)
        bufs = ring[:SUBS]
        sems = ring[SUBS:]

        sp_in = pltpu.make_async_copy(
            spatial_hbm.at[pl.ds(s_base, rows), :], sp_v, sems[0])
        tq_in = pltpu.make_async_copy(
            temporal_hbm.at[pl.ds(t_base, tpg), :], tq_v, sems[1])
        sp_in.start()
        tq_in.start()
        tq_in.wait()
        sp_in.wait()

        def compute(tl, sub, buf):
            @plsc.parallel_loop(0, dv, unroll=3)
            def body(j):
                col = pl.multiple_of(j * LANES, LANES)
                tvec = tq_v[tl, pl.ds(col, LANES)]
                for r in range(srows):
                    buf[r, pl.ds(col, LANES)] = (
                        sp_v[sub * srows + r, pl.ds(col, LANES)] + tvec
                    )

        def out_dma(tl, sub, buf, sem):
            s_off = pl.multiple_of(s_base + sub * srows, 8)
            return pltpu.make_async_copy(
                buf, out_hbm.at[t_base + tl, pl.ds(s_off, srows), :], sem
            )

        for sub in range(SUBS):
            compute(0, sub, bufs[sub])
            out_dma(0, sub, bufs[sub], sems[sub]).start()

        @pl.loop(1, tpg)
        def t_loop(tl):
            for sub in range(SUBS):
                out_dma(tl - 1, sub, bufs[sub], sems[sub]).wait()
                compute(tl, sub, bufs[sub])
                out_dma(tl, sub, bufs[sub], sems[sub]).start()

        for sub in range(SUBS):
            out_dma(tpg - 1, sub, bufs[sub], sems[sub]).wait()

    return k


@jax.jit
def kernel(x, spatial_table, temporal_table):
    T = x.shape[1]
    S = x.shape[2]
    D = spatial_table.shape[1]
    k = _make_sc_kernel(T, S, D)
    return k(spatial_table[:S].astype(jnp.float32),
             temporal_table.astype(jnp.float32))
